# Initial kernel scaffold; baseline (speedup 1.0000x reference)
import jax
import jax.numpy as jnp
from jax.experimental import pallas as pl


def kernel(center_emb, all_embs, neighbor_indices, neighbor_weights, Wq, Wk, Wg, bg):
    raise NotImplementedError("write your pallas kernel here")


# speedup vs baseline: 1.5644x; 1.5644x over previous
import dataclasses
import functools

import jax
import jax.numpy as jnp
from jax import lax
from jax.experimental import pallas as pl
from jax.experimental.pallas import tpu as pltpu
from jax.experimental.pallas import tpu_sc as plsc

B = 16384
N = 100000
D = 256
K = 16
A = 64

NW = 32
CPW = B // NW
CHUNK = 64
NCHUNK = CPW // CHUNK
GB = 8
GROWS = GB * K
NBATCH = CHUNK // GB

_NEG = -1e30


def _tc_pre_body(cb_ref, w_ref, wq_ref, wkt_ref, p_ref, slog_ref):
    q = jnp.dot(cb_ref[...], wq_ref[...], preferred_element_type=jnp.float32)
    p_ref[...] = jnp.dot(q, wkt_ref[...], preferred_element_type=jnp.float32)
    w = w_ref[...]
    slog_ref[...] = jnp.where(w < 1e-6, _NEG, jnp.log(jnp.maximum(w, 1e-6)))


def _tc_post_body(cb_ref, wn_ref, wg1_ref, wg2_ref, bg_ref, o_ref):
    cb = cb_ref[...]
    wn = wn_ref[...]
    z = (jnp.dot(cb, wg1_ref[...], preferred_element_type=jnp.float32)
         + jnp.dot(wn, wg2_ref[...], preferred_element_type=jnp.float32)
         + bg_ref[...])
    g = jax.nn.sigmoid(z)
    o_ref[...] = g * cb + (1.0 - g) * wn


def _sc_attention(all_embs, idx2, p, slog):
    mesh = plsc.VectorSubcoreMesh(core_axis_name="c", subcore_axis_name="s")
    cp = pltpu.CompilerParams()
    if "needs_layout_passes" in pltpu.CompilerParams.__dataclass_fields__:
        cp = dataclasses.replace(cp, needs_layout_passes=False)

    @functools.partial(
        pl.kernel,
        out_type=jax.ShapeDtypeStruct((B, D), jnp.float32),
        mesh=mesh,
        compiler_params=cp,
        scratch_types=[
            pltpu.VMEM((NBATCH, GROWS), jnp.int32),
            pltpu.VMEM((CHUNK, D), jnp.float32),
            pltpu.VMEM((CHUNK, K), jnp.float32),
            pltpu.VMEM((CHUNK, D), jnp.float32),
            pltpu.VMEM((GROWS, D), jnp.float32),
            pltpu.VMEM((GROWS, D), jnp.float32),
            pltpu.SemaphoreType.DMA,
            pltpu.SemaphoreType.DMA,
        ],
    )
    def sc_kernel(embs_hbm, idx_hbm, p_hbm, slog_hbm, wn_hbm,
                  idx_v, p_v, slog_v, out_v, bufA, bufB, semA, semB):
        cid = lax.axis_index("c")
        sid = lax.axis_index("s")
        wid = sid * 2 + cid
        lane = lax.broadcasted_iota(jnp.int32, (K,), 0)

        def compute_batch(g, buf):
            @pl.loop(0, GB)
            def _t(t):
                tl = g * GB + t
                r0 = t * K
                pch = [p_v[tl, pl.ds(cc * 16, 16)] for cc in range(16)]
                s = slog_v[tl, :]
                for j in range(K):
                    acc = pch[0] * buf[r0 + j, pl.ds(0, 16)]
                    for cc in range(1, 16):
                        acc = acc + pch[cc] * buf[r0 + j, pl.ds(cc * 16, 16)]
                    sj = jnp.sum(acc)
                    s = jnp.where(lane == j, s + sj, s)
                m = jnp.max(s)
                e = jnp.exp(s - m)
                e = jnp.where(s > -1e29, e, 0.0)
                den = jnp.sum(e)
                den = jnp.where(den > 0.0, den, 1.0)
                attn = e / den
                aj = [attn[j] for j in range(K)]
                for cc in range(16):
                    acc = aj[0] * buf[r0, pl.ds(cc * 16, 16)]
                    for j in range(1, K):
                        acc = acc + aj[j] * buf[r0 + j, pl.ds(cc * 16, 16)]
                    out_v[tl, pl.ds(cc * 16, 16)] = acc

        @pl.loop(0, NCHUNK)
        def _chunk(c):
            cbase = pl.multiple_of(wid * CPW + c * CHUNK, CHUNK)
            irow = pl.multiple_of(wid * (CPW * K // GROWS) + c * NBATCH, NBATCH)
            pltpu.sync_copy(idx_hbm.at[pl.ds(irow, NBATCH)], idx_v)
            pltpu.sync_copy(p_hbm.at[pl.ds(cbase, CHUNK)], p_v)
            pltpu.sync_copy(slog_hbm.at[pl.ds(cbase, CHUNK)], slog_v)
            pltpu.make_async_copy(embs_hbm.at[idx_v.at[0]], bufA, semA).start()

            @pl.loop(0, NBATCH, step=2)
            def _g(g):
                pltpu.make_async_copy(
                    embs_hbm.at[idx_v.at[g + 1]], bufB, semB).start()
                pltpu.make_async_copy(
                    embs_hbm.at[idx_v.at[g]], bufA, semA).wait()
                compute_batch(g, bufA)

                @pl.when(g + 2 < NBATCH)
                def _():
                    pltpu.make_async_copy(
                        embs_hbm.at[idx_v.at[g + 2]], bufA, semA).start()

                pltpu.make_async_copy(
                    embs_hbm.at[idx_v.at[g + 1]], bufB, semB).wait()
                compute_batch(g + 1, bufB)

            pltpu.sync_copy(out_v, wn_hbm.at[pl.ds(cbase, CHUNK)])

    return sc_kernel(all_embs, idx2, p, slog)


def kernel(center_emb, all_embs, neighbor_indices, neighbor_weights, Wq, Wk, Wg, bg):
    scale = A ** (-0.5)
    wkt = (Wk.T * scale).astype(jnp.float32)
    wg1 = Wg[:D]
    wg2 = Wg[D:]
    bg2 = bg.reshape(1, D)
    idx2 = neighbor_indices.astype(jnp.int32).reshape(B * K // GROWS, GROWS)

    bb = 2048
    p, slog = pl.pallas_call(
        _tc_pre_body,
        grid=(B // bb,),
        in_specs=[
            pl.BlockSpec((bb, D), lambda i: (i, 0)),
            pl.BlockSpec((bb, K), lambda i: (i, 0)),
            pl.BlockSpec((D, A), lambda i: (0, 0)),
            pl.BlockSpec((A, D), lambda i: (0, 0)),
        ],
        out_specs=[
            pl.BlockSpec((bb, D), lambda i: (i, 0)),
            pl.BlockSpec((bb, K), lambda i: (i, 0)),
        ],
        out_shape=[
            jax.ShapeDtypeStruct((B, D), jnp.float32),
            jax.ShapeDtypeStruct((B, K), jnp.float32),
        ],
    )(center_emb, neighbor_weights, Wq, wkt)

    wn = _sc_attention(all_embs, idx2, p, slog)

    out = pl.pallas_call(
        _tc_post_body,
        grid=(B // bb,),
        in_specs=[
            pl.BlockSpec((bb, D), lambda i: (i, 0)),
            pl.BlockSpec((bb, D), lambda i: (i, 0)),
            pl.BlockSpec((D, D), lambda i: (0, 0)),
            pl.BlockSpec((D, D), lambda i: (0, 0)),
            pl.BlockSpec((1, D), lambda i: (0, 0)),
        ],
        out_specs=pl.BlockSpec((bb, D), lambda i: (i, 0)),
        out_shape=jax.ShapeDtypeStruct((B, D), jnp.float32),
    )(center_emb, wn, wg1, wg2, bg2)
    return out
